# Initial kernel scaffold; baseline (speedup 1.0000x reference)
import jax
import jax.numpy as jnp
from jax.experimental import pallas as pl


def kernel(x, edge_index, edge_attr, batch, W1, as1, ad1, ae1, We1, b1, W2, as2, ad2, ae2, We2, b2, ln_g, ln_b, fc_w, fc_b):
    raise NotImplementedError("write your pallas kernel here")


# speedup vs baseline: 12.0859x; 12.0859x over previous
import functools

import jax
import jax.numpy as jnp
from jax import lax
from jax.experimental import pallas as pl
from jax.experimental.pallas import tpu as pltpu
from jax.experimental.pallas import tpu_sc as plsc

N = 10000
E = 320000
D = 128
H = 128
DE = 16
G = 64
ET = E + N
NC = 2
NS = 16
NW = NC * NS
PW = 10368
EP = NW * PW
NP = 10240
K1 = 192
K2 = 128
EB = 4000
XB = 1000

_f32 = jnp.float32
_i32 = jnp.int32


def _edge_pre_body(ea_ref, we1_ref, ae1_ref, we2_ref, ae2_ref,
                   ce1_ref, ce2_ref, st_ref):
    ea = ea_ref[...]
    ve1 = jnp.dot(we1_ref[...], ae1_ref[...], preferred_element_type=_f32)
    ve2 = jnp.dot(we2_ref[...], ae2_ref[...], preferred_element_type=_f32)
    ce1 = jnp.dot(ea, ve1, preferred_element_type=_f32)
    ce2 = jnp.dot(ea, ve2, preferred_element_type=_f32)
    ce1_ref[...] = ce1
    ce2_ref[...] = ce2
    st_ref[...] = jnp.concatenate(
        [jnp.sum(ce1).reshape(1, 1, 1), jnp.max(ce1).reshape(1, 1, 1),
         jnp.sum(ce2).reshape(1, 1, 1), jnp.max(ce2).reshape(1, 1, 1)],
        axis=2)


def _edge_pre(ea, we1, ae1, we2, ae2):
    nb = E // EB
    return pl.pallas_call(
        _edge_pre_body,
        grid=(nb,),
        in_specs=[
            pl.BlockSpec((EB, DE), lambda i: (i, 0)),
            pl.BlockSpec((DE, H), lambda i: (0, 0)),
            pl.BlockSpec((H, 1), lambda i: (0, 0)),
            pl.BlockSpec((DE, H), lambda i: (0, 0)),
            pl.BlockSpec((H, 1), lambda i: (0, 0)),
        ],
        out_specs=[
            pl.BlockSpec((EB, 1), lambda i: (i, 0)),
            pl.BlockSpec((EB, 1), lambda i: (i, 0)),
            pl.BlockSpec((1, 1, 4), lambda i: (i, 0, 0)),
        ],
        out_shape=[
            jax.ShapeDtypeStruct((E, 1), _f32),
            jax.ShapeDtypeStruct((E, 1), _f32),
            jax.ShapeDtypeStruct((nb, 1, 4), _f32),
        ],
    )(ea, we1, ae1, we2, ae2)


def _node_pre_body(x_ref, w_ref, as_ref, ad_ref, h_ref, s_ref, d_ref, st_ref):
    h = jnp.dot(x_ref[...], w_ref[...], preferred_element_type=_f32)
    s = jnp.dot(h, as_ref[...], preferred_element_type=_f32)
    d = jnp.dot(h, ad_ref[...], preferred_element_type=_f32)
    h_ref[...] = h
    s_ref[...] = s
    d_ref[...] = d
    st_ref[...] = jnp.concatenate(
        [jnp.max(s).reshape(1, 1, 1), jnp.max(d).reshape(1, 1, 1)], axis=2)


def _node_pre(x, w, a_s, a_d):
    nb = N // XB
    return pl.pallas_call(
        _node_pre_body,
        grid=(nb,),
        in_specs=[
            pl.BlockSpec((XB, D), lambda i: (i, 0)),
            pl.BlockSpec((D, H), lambda i: (0, 0)),
            pl.BlockSpec((H, 1), lambda i: (0, 0)),
            pl.BlockSpec((H, 1), lambda i: (0, 0)),
        ],
        out_specs=[
            pl.BlockSpec((XB, H), lambda i: (i, 0)),
            pl.BlockSpec((XB, 1), lambda i: (i, 0)),
            pl.BlockSpec((XB, 1), lambda i: (i, 0)),
            pl.BlockSpec((1, 1, 2), lambda i: (i, 0, 0)),
        ],
        out_shape=[
            jax.ShapeDtypeStruct((N, H), _f32),
            jax.ShapeDtypeStruct((N, 1), _f32),
            jax.ShapeDtypeStruct((N, 1), _f32),
            jax.ShapeDtypeStruct((nb, 1, 2), _f32),
        ],
    )(x, w, a_s, a_d)


def _ln_lrelu(o, g_ref, bl_ref):
    mu = jnp.mean(o, axis=-1, keepdims=True)
    c = o - mu
    var = jnp.mean(c * c, axis=-1, keepdims=True)
    t = c * lax.rsqrt(var + 1e-5) * g_ref[...] + bl_ref[...]
    return jnp.where(t >= 0, t, t * 0.1)


def _post1_body(o0_ref, o1_ref, b_ref, g_ref, bl_ref, w_ref, as_ref, ad_ref,
                h_ref, s_ref, d_ref, st_ref):
    o = o0_ref[...] + o1_ref[...] + b_ref[...]
    t = _ln_lrelu(o, g_ref, bl_ref)
    h = jnp.dot(t, w_ref[...], preferred_element_type=_f32)
    s = jnp.dot(h, as_ref[...], preferred_element_type=_f32)
    d = jnp.dot(h, ad_ref[...], preferred_element_type=_f32)
    h_ref[...] = h
    s_ref[...] = s
    d_ref[...] = d
    st_ref[...] = jnp.concatenate(
        [jnp.max(s).reshape(1, 1, 1), jnp.max(d).reshape(1, 1, 1)], axis=2)


def _post1(o0, o1, b, g, bl, w, a_s, a_d):
    nb = N // XB
    return pl.pallas_call(
        _post1_body,
        grid=(nb,),
        in_specs=[
            pl.BlockSpec((XB, H), lambda i: (i, 0)),
            pl.BlockSpec((XB, H), lambda i: (i, 0)),
            pl.BlockSpec((1, H), lambda i: (0, 0)),
            pl.BlockSpec((1, H), lambda i: (0, 0)),
            pl.BlockSpec((1, H), lambda i: (0, 0)),
            pl.BlockSpec((H, H), lambda i: (0, 0)),
            pl.BlockSpec((H, 1), lambda i: (0, 0)),
            pl.BlockSpec((H, 1), lambda i: (0, 0)),
        ],
        out_specs=[
            pl.BlockSpec((XB, H), lambda i: (i, 0)),
            pl.BlockSpec((XB, 1), lambda i: (i, 0)),
            pl.BlockSpec((XB, 1), lambda i: (i, 0)),
            pl.BlockSpec((1, 1, 2), lambda i: (i, 0, 0)),
        ],
        out_shape=[
            jax.ShapeDtypeStruct((N, H), _f32),
            jax.ShapeDtypeStruct((N, 1), _f32),
            jax.ShapeDtypeStruct((N, 1), _f32),
            jax.ShapeDtypeStruct((nb, 1, 2), _f32),
        ],
    )(o0, o1, b, g, bl, w, a_s, a_d)


def _post2_body(o0_ref, o1_ref, b_ref, g_ref, bl_ref, hr_ref):
    o = o0_ref[...] + o1_ref[...] + b_ref[...]
    t = _ln_lrelu(o, g_ref, bl_ref)
    hr_ref[...] = jnp.maximum(t, 0.0)


def _post2(o0, o1, b, g, bl):
    nb = N // XB
    return pl.pallas_call(
        _post2_body,
        grid=(nb,),
        in_specs=[
            pl.BlockSpec((XB, H), lambda i: (i, 0)),
            pl.BlockSpec((XB, H), lambda i: (i, 0)),
            pl.BlockSpec((1, H), lambda i: (0, 0)),
            pl.BlockSpec((1, H), lambda i: (0, 0)),
            pl.BlockSpec((1, H), lambda i: (0, 0)),
        ],
        out_specs=pl.BlockSpec((XB, H), lambda i: (i, 0)),
        out_shape=jax.ShapeDtypeStruct((N, H), _f32),
    )(o0, o1, b, g, bl)


def _pool_final_body(pp_ref, fw_ref, fb_ref, o_ref):
    m = jnp.max(pp_ref[...], axis=0)
    o_ref[...] = jnp.dot(m, fw_ref[...], preferred_element_type=_f32) \
        + fb_ref[...]


def _pool_final(pp, fw, fb):
    return pl.pallas_call(
        _pool_final_body,
        grid=(G // 8,),
        in_specs=[
            pl.BlockSpec((NW, 8, H), lambda i: (0, i, 0)),
            pl.BlockSpec((H, 1), lambda i: (0, 0)),
            pl.BlockSpec((1, 1), lambda i: (0, 0)),
        ],
        out_specs=pl.BlockSpec((8, 1), lambda i: (i, 0)),
        out_shape=jax.ShapeDtypeStruct((G, 1), _f32),
    )(pp, fw, fb)


def _sc_mesh():
    return plsc.VectorSubcoreMesh(core_axis_name="c", subcore_axis_name="s")


_SC_PARAMS = pltpu.CompilerParams(needs_layout_passes=False)


@functools.partial(
    pl.kernel,
    out_type=[
        jax.ShapeDtypeStruct((EP,), _f32),
        jax.ShapeDtypeStruct((NW * NP,), _f32),
    ],
    mesh=_sc_mesh(),
    compiler_params=_SC_PARAMS,
    scratch_types=[
        pltpu.VMEM((NP,), _f32),
        pltpu.VMEM((NP,), _f32),
        pltpu.VMEM((NP,), _f32),
        pltpu.VMEM((16,), _f32),
        pltpu.VMEM((K1,), _i32),
        pltpu.VMEM((K1,), _i32),
        pltpu.VMEM((K1,), _f32),
        pltpu.VMEM((K1,), _f32),
    ],
)
def _sc_pass1(s_hbm, d_hbm, ce_hbm, src_hbm, dst_hbm, c_hbm,
              p_hbm, sspart_hbm,
              s_t, d_t, ssum_t, c_t, src_t, dst_t, ce_t, p_t):
    cid = lax.axis_index("c")
    sid = lax.axis_index("s")
    wid = cid * NS + sid
    pltpu.sync_copy(s_hbm, s_t)
    pltpu.sync_copy(d_hbm, d_t)
    pltpu.sync_copy(c_hbm, c_t)
    cv = c_t[...]

    def zero(i, carry):
        ssum_t[pl.ds(i * 16, 16)] = jnp.zeros((16,), _f32)
        return carry
    lax.fori_loop(0, NP // 16, zero, 0)

    def step(t, carry):
        base = wid * PW + t * K1
        pltpu.sync_copy(src_hbm.at[pl.ds(base, K1)], src_t)
        pltpu.sync_copy(dst_hbm.at[pl.ds(base, K1)], dst_t)
        pltpu.sync_copy(ce_hbm.at[pl.ds(base, K1)], ce_t)
        for j in range(K1 // 16):
            sl = pl.ds(j * 16, 16)
            si = src_t[sl]
            di = dst_t[sl]
            a = plsc.load_gather(s_t, [si]) + plsc.load_gather(d_t, [di]) \
                + ce_t[sl]
            a = jnp.where(a >= 0, a, a * 0.2)
            pv = jnp.exp(a - cv)
            p_t[sl] = pv
            plsc.addupdate_scatter(ssum_t, [di], pv)
        pltpu.sync_copy(p_t, p_hbm.at[pl.ds(base, K1)])
        return carry
    lax.fori_loop(0, PW // K1, step, 0)
    pltpu.sync_copy(ssum_t, sspart_hbm.at[pl.ds(wid * NP, NP)])


@functools.partial(
    pl.kernel,
    out_type=[
        jax.ShapeDtypeStruct((EP,), _f32),
        jax.ShapeDtypeStruct((NC, NP, H), _f32),
    ],
    mesh=_sc_mesh(),
    compiler_params=_SC_PARAMS,
    scratch_types=[
        pltpu.VMEM((NP,), _f32),
        pltpu.VMEM((NW * NP // NS // 2,), _f32),
        pltpu.VMEM((NP // NS // 2,), _f32),
        pltpu.VMEM((K2,), _i32),
        pltpu.VMEM((K2,), _i32),
        pltpu.VMEM((K2,), _f32),
        pltpu.VMEM((K2,), _f32),
        pltpu.VMEM((K2, H), _f32),
        pltpu.VMEM_SHARED((NP,), _f32),
        pltpu.VMEM_SHARED((NP, H), _f32),
        pltpu.SemaphoreType.DMA,
    ],
)
def _sc_pass2(sspart_hbm, p_hbm, src_hbm, dst_hbm, h_hbm, z_hbm,
              w_hbm, outp_hbm,
              ssum_t, red_t, str_t, src_t, dst_t, p_t, w_t, rows_t,
              ssum_sh, acc_sh, sem):
    cid = lax.axis_index("c")
    sid = lax.axis_index("s")
    wid = cid * NS + sid
    stripe = NP // NS
    half = stripe // 2

    for hh in range(2):
        off = sid * stripe + hh * half
        for j in range(NW):
            pltpu.sync_copy(sspart_hbm.at[pl.ds(j * NP + off, half)],
                            red_t.at[pl.ds(j * half, half)])

        def red(g, carry):
            acc = red_t[pl.ds(g * 16, 16)]
            for j in range(1, NW):
                acc = acc + red_t[pl.ds(j * half + g * 16, 16)]
            str_t[pl.ds(g * 16, 16)] = acc
            return carry
        lax.fori_loop(0, half // 16, red, 0)
        pltpu.sync_copy(str_t, ssum_sh.at[pl.ds(off, half)])

    for i in range(stripe // 64):
        pltpu.sync_copy(z_hbm, acc_sh.at[pl.ds(sid * stripe + i * 64, 64)])
    plsc.subcore_barrier()
    pltpu.sync_copy(ssum_sh, ssum_t)

    def step(t, carry):
        base = wid * PW + t * K2
        pltpu.sync_copy(src_hbm.at[pl.ds(base, K2)], src_t)
        pltpu.sync_copy(dst_hbm.at[pl.ds(base, K2)], dst_t)
        pltpu.sync_copy(p_hbm.at[pl.ds(base, K2)], p_t)
        pltpu.async_copy(h_hbm.at[src_t], rows_t, sem).wait()
        for j in range(K2 // 16):
            sl = pl.ds(j * 16, 16)
            di = dst_t[sl]
            sv = plsc.load_gather(ssum_t, [di])
            w_t[sl] = p_t[sl] / (sv + 1e-16)
        pltpu.sync_copy(w_t, w_hbm.at[pl.ds(base, K2)])
        for r in range(K2):
            wb = plsc.load_gather(w_t, [jnp.full((16,), r, _i32)])
            for c in range(H // 16):
                cs = pl.ds(c * 16, 16)
                rows_t[r, cs] = rows_t[r, cs] * wb
        pltpu.sync_copy(rows_t, acc_sh.at[dst_t], add=True)
        return carry
    lax.fori_loop(0, PW // K2, step, 0)

    plsc.subcore_barrier()
    for i in range(stripe // 64):
        row0 = sid * stripe + i * 64
        pltpu.sync_copy(acc_sh.at[pl.ds(row0, 64)], rows_t.at[pl.ds(0, 64)])
        pltpu.sync_copy(rows_t.at[pl.ds(0, 64)],
                        outp_hbm.at[cid, pl.ds(row0, 64)])


@functools.partial(
    pl.kernel,
    out_type=jax.ShapeDtypeStruct((NW, G, H), _f32),
    mesh=_sc_mesh(),
    compiler_params=_SC_PARAMS,
    scratch_types=[
        pltpu.VMEM((G, H), _f32),
        pltpu.VMEM((64, H), _f32),
        pltpu.VMEM((NP // NW,), _i32),
    ],
)
def _sc_pool(hr_hbm, bat_hbm, z_hbm, pool_hbm, tbl_t, rows_t, bat_t):
    cid = lax.axis_index("c")
    sid = lax.axis_index("s")
    wid = cid * NS + sid
    rpw = NP // NW
    pltpu.sync_copy(z_hbm, tbl_t)
    pltpu.sync_copy(bat_hbm.at[pl.ds(wid * rpw, rpw)], bat_t)
    colio = lax.iota(_i32, 16)

    def chunk(i, carry):
        pltpu.sync_copy(hr_hbm.at[pl.ds(wid * rpw + i * 64, 64)], rows_t)

        def row(r, carry2):
            gv = plsc.load_gather(bat_t, [jnp.full((16,), i * 64, _i32) + r])
            ri = jnp.full((16,), r, _i32)
            for c in range(H // 16):
                ci = colio + (c * 16)
                v = plsc.load_gather(rows_t, [ri, ci])
                cur = plsc.load_gather(tbl_t, [gv, ci])
                plsc.store_scatter(tbl_t, [gv, ci], jnp.maximum(cur, v))
            return carry2
        lax.fori_loop(0, 64, row, 0)
        return carry
    lax.fori_loop(0, rpw // 64, chunk, 0)
    pltpu.sync_copy(tbl_t, pool_hbm.at[wid])


def kernel(x, edge_index, edge_attr, batch, W1, as1, ad1, ae1, We1, b1,
           W2, as2, ad2, ae2, We2, b2, ln_g, ln_b, fc_w, fc_b):
    loop = jnp.arange(N, dtype=_i32)
    padi = jnp.zeros((EP - ET,), _i32)
    src = jnp.concatenate([edge_index[0].astype(_i32), loop, padi])
    dst = jnp.concatenate([edge_index[1].astype(_i32), loop, padi])

    ce1e, ce2e, est = _edge_pre(edge_attr, We1, ae1.reshape(H, 1),
                                We2, ae2.reshape(H, 1))
    mean1 = jnp.sum(est[:, 0, 0]) / E
    mean2 = jnp.sum(est[:, 0, 2]) / E
    maxce1 = jnp.maximum(jnp.max(est[:, 0, 1]), mean1)
    maxce2 = jnp.maximum(jnp.max(est[:, 0, 3]), mean2)
    padf = jnp.full((EP - ET,), -1e30, _f32)
    ce1 = jnp.concatenate([ce1e.reshape(-1), jnp.full((N,), mean1, _f32), padf])
    ce2 = jnp.concatenate([ce2e.reshape(-1), jnp.full((N,), mean2, _f32), padf])

    zrows = jnp.zeros((64, H), _f32)
    b1r = b1.reshape(1, H)
    b2r = b2.reshape(1, H)
    gr = ln_g.reshape(1, H)
    blr = ln_b.reshape(1, H)

    def _padn(v):
        return jnp.concatenate([v.reshape(-1), jnp.zeros((NP - N,), _f32)])

    h1, s1, d1, nst1 = _node_pre(x, W1, as1.reshape(H, 1), ad1.reshape(H, 1))
    c1 = jnp.maximum(jnp.max(nst1[:, 0, 0]) + jnp.max(nst1[:, 0, 1]) + maxce1,
                     0.0)
    p1, sspart1 = _sc_pass1(_padn(s1), _padn(d1), ce1, src, dst,
                            jnp.full((16,), c1, _f32))
    w1, outp1 = _sc_pass2(sspart1, p1, src, dst, h1, zrows)

    h2, s2, d2, nst2 = _post1(outp1[0, :N], outp1[1, :N], b1r, gr, blr,
                              W2, as2.reshape(H, 1), ad2.reshape(H, 1))
    c2 = jnp.maximum(jnp.max(nst2[:, 0, 0]) + jnp.max(nst2[:, 0, 1]) + maxce2,
                     0.0)
    p2, sspart2 = _sc_pass1(_padn(s2), _padn(d2), ce2, src, dst,
                            jnp.full((16,), c2, _f32))
    w2, outp2 = _sc_pass2(sspart2, p2, src, dst, h2, zrows)

    hrel = _post2(outp2[0, :N], outp2[1, :N], b2r, gr, blr)
    hrelp = jnp.concatenate([hrel, jnp.zeros((NP - N, H), _f32)], axis=0)
    batp = jnp.concatenate([batch.astype(_i32), jnp.zeros((NP - N,), _i32)])
    pool = _sc_pool(hrelp, batp, zrows)
    out = _pool_final(pool, fc_w, fc_b.reshape(1, 1))
    return (out.reshape(-1), w1[:ET], w2[:ET])


# speedup vs baseline: 20.0278x; 1.6571x over previous
import functools

import jax
import jax.numpy as jnp
from jax import lax
from jax.experimental import pallas as pl
from jax.experimental.pallas import tpu as pltpu
from jax.experimental.pallas import tpu_sc as plsc

N = 10000
E = 320000
D = 128
H = 128
DE = 16
G = 64
ET = E + N
NC = 2
NS = 16
NW = NC * NS
PW = 10368
EP = NW * PW
NP = 10240
K1 = 192
K2 = 96
SUP = 18 * K2
EB = 4000
XB = 1000

_f32 = jnp.float32
_i32 = jnp.int32


def _edge_pre_body(ea_ref, we1_ref, ae1_ref, we2_ref, ae2_ref,
                   ce1_ref, ce2_ref, st_ref):
    ea = ea_ref[...]
    ve1 = jnp.dot(we1_ref[...], ae1_ref[...], preferred_element_type=_f32)
    ve2 = jnp.dot(we2_ref[...], ae2_ref[...], preferred_element_type=_f32)
    ce1 = jnp.dot(ea, ve1, preferred_element_type=_f32)
    ce2 = jnp.dot(ea, ve2, preferred_element_type=_f32)
    ce1_ref[...] = ce1
    ce2_ref[...] = ce2
    st_ref[...] = jnp.concatenate(
        [jnp.sum(ce1).reshape(1, 1, 1), jnp.max(ce1).reshape(1, 1, 1),
         jnp.sum(ce2).reshape(1, 1, 1), jnp.max(ce2).reshape(1, 1, 1)],
        axis=2)


def _edge_pre(ea, we1, ae1, we2, ae2):
    nb = E // EB
    return pl.pallas_call(
        _edge_pre_body,
        grid=(nb,),
        in_specs=[
            pl.BlockSpec((EB, DE), lambda i: (i, 0)),
            pl.BlockSpec((DE, H), lambda i: (0, 0)),
            pl.BlockSpec((H, 1), lambda i: (0, 0)),
            pl.BlockSpec((DE, H), lambda i: (0, 0)),
            pl.BlockSpec((H, 1), lambda i: (0, 0)),
        ],
        out_specs=[
            pl.BlockSpec((EB, 1), lambda i: (i, 0)),
            pl.BlockSpec((EB, 1), lambda i: (i, 0)),
            pl.BlockSpec((1, 1, 4), lambda i: (i, 0, 0)),
        ],
        out_shape=[
            jax.ShapeDtypeStruct((E, 1), _f32),
            jax.ShapeDtypeStruct((E, 1), _f32),
            jax.ShapeDtypeStruct((nb, 1, 4), _f32),
        ],
    )(ea, we1, ae1, we2, ae2)


def _node_pre_body(x_ref, w_ref, as_ref, ad_ref, h_ref, s_ref, d_ref, st_ref):
    h = jnp.dot(x_ref[...], w_ref[...], preferred_element_type=_f32)
    s = jnp.dot(h, as_ref[...], preferred_element_type=_f32)
    d = jnp.dot(h, ad_ref[...], preferred_element_type=_f32)
    h_ref[...] = h
    s_ref[...] = s
    d_ref[...] = d
    st_ref[...] = jnp.concatenate(
        [jnp.max(s).reshape(1, 1, 1), jnp.max(d).reshape(1, 1, 1)], axis=2)


def _node_pre(x, w, a_s, a_d):
    nb = N // XB
    return pl.pallas_call(
        _node_pre_body,
        grid=(nb,),
        in_specs=[
            pl.BlockSpec((XB, D), lambda i: (i, 0)),
            pl.BlockSpec((D, H), lambda i: (0, 0)),
            pl.BlockSpec((H, 1), lambda i: (0, 0)),
            pl.BlockSpec((H, 1), lambda i: (0, 0)),
        ],
        out_specs=[
            pl.BlockSpec((XB, H), lambda i: (i, 0)),
            pl.BlockSpec((XB, 1), lambda i: (i, 0)),
            pl.BlockSpec((XB, 1), lambda i: (i, 0)),
            pl.BlockSpec((1, 1, 2), lambda i: (i, 0, 0)),
        ],
        out_shape=[
            jax.ShapeDtypeStruct((N, H), _f32),
            jax.ShapeDtypeStruct((N, 1), _f32),
            jax.ShapeDtypeStruct((N, 1), _f32),
            jax.ShapeDtypeStruct((nb, 1, 2), _f32),
        ],
    )(x, w, a_s, a_d)


def _ln_lrelu(o, g_ref, bl_ref):
    mu = jnp.mean(o, axis=-1, keepdims=True)
    c = o - mu
    var = jnp.mean(c * c, axis=-1, keepdims=True)
    t = c * lax.rsqrt(var + 1e-5) * g_ref[...] + bl_ref[...]
    return jnp.where(t >= 0, t, t * 0.1)


def _post1_body(o0_ref, o1_ref, b_ref, g_ref, bl_ref, w_ref, as_ref, ad_ref,
                h_ref, s_ref, d_ref, st_ref):
    o = o0_ref[...] + o1_ref[...] + b_ref[...]
    t = _ln_lrelu(o, g_ref, bl_ref)
    h = jnp.dot(t, w_ref[...], preferred_element_type=_f32)
    s = jnp.dot(h, as_ref[...], preferred_element_type=_f32)
    d = jnp.dot(h, ad_ref[...], preferred_element_type=_f32)
    h_ref[...] = h
    s_ref[...] = s
    d_ref[...] = d
    st_ref[...] = jnp.concatenate(
        [jnp.max(s).reshape(1, 1, 1), jnp.max(d).reshape(1, 1, 1)], axis=2)


def _post1(o0, o1, b, g, bl, w, a_s, a_d):
    nb = N // XB
    return pl.pallas_call(
        _post1_body,
        grid=(nb,),
        in_specs=[
            pl.BlockSpec((XB, H), lambda i: (i, 0)),
            pl.BlockSpec((XB, H), lambda i: (i, 0)),
            pl.BlockSpec((1, H), lambda i: (0, 0)),
            pl.BlockSpec((1, H), lambda i: (0, 0)),
            pl.BlockSpec((1, H), lambda i: (0, 0)),
            pl.BlockSpec((H, H), lambda i: (0, 0)),
            pl.BlockSpec((H, 1), lambda i: (0, 0)),
            pl.BlockSpec((H, 1), lambda i: (0, 0)),
        ],
        out_specs=[
            pl.BlockSpec((XB, H), lambda i: (i, 0)),
            pl.BlockSpec((XB, 1), lambda i: (i, 0)),
            pl.BlockSpec((XB, 1), lambda i: (i, 0)),
            pl.BlockSpec((1, 1, 2), lambda i: (i, 0, 0)),
        ],
        out_shape=[
            jax.ShapeDtypeStruct((N, H), _f32),
            jax.ShapeDtypeStruct((N, 1), _f32),
            jax.ShapeDtypeStruct((N, 1), _f32),
            jax.ShapeDtypeStruct((nb, 1, 2), _f32),
        ],
    )(o0, o1, b, g, bl, w, a_s, a_d)


def _post2_body(o0_ref, o1_ref, b_ref, g_ref, bl_ref, hr_ref):
    o = o0_ref[...] + o1_ref[...] + b_ref[...]
    t = _ln_lrelu(o, g_ref, bl_ref)
    hr_ref[...] = jnp.maximum(t, 0.0)


def _post2(o0, o1, b, g, bl):
    nb = N // XB
    return pl.pallas_call(
        _post2_body,
        grid=(nb,),
        in_specs=[
            pl.BlockSpec((XB, H), lambda i: (i, 0)),
            pl.BlockSpec((XB, H), lambda i: (i, 0)),
            pl.BlockSpec((1, H), lambda i: (0, 0)),
            pl.BlockSpec((1, H), lambda i: (0, 0)),
            pl.BlockSpec((1, H), lambda i: (0, 0)),
        ],
        out_specs=pl.BlockSpec((XB, H), lambda i: (i, 0)),
        out_shape=jax.ShapeDtypeStruct((N, H), _f32),
    )(o0, o1, b, g, bl)


def _pool_final_body(pp_ref, fw_ref, fb_ref, o_ref):
    m = jnp.max(pp_ref[...], axis=0)
    o_ref[...] = jnp.dot(m, fw_ref[...], preferred_element_type=_f32) \
        + fb_ref[...]


def _pool_final(pp, fw, fb):
    return pl.pallas_call(
        _pool_final_body,
        grid=(G // 8,),
        in_specs=[
            pl.BlockSpec((NW, 8, H), lambda i: (0, i, 0)),
            pl.BlockSpec((H, 1), lambda i: (0, 0)),
            pl.BlockSpec((1, 1), lambda i: (0, 0)),
        ],
        out_specs=pl.BlockSpec((8, 1), lambda i: (i, 0)),
        out_shape=jax.ShapeDtypeStruct((G, 1), _f32),
    )(pp, fw, fb)


def _ssum_reduce_body(pp_ref, o_ref):
    o_ref[...] = jnp.sum(pp_ref[...], axis=0, keepdims=True)[None]


def _ssum_reduce(pp):
    return pl.pallas_call(
        _ssum_reduce_body,
        grid=(8,),
        in_specs=[pl.BlockSpec((NW, NP // 8), lambda i: (0, i))],
        out_specs=pl.BlockSpec((1, 1, NP // 8), lambda i: (i, 0, 0)),
        out_shape=jax.ShapeDtypeStruct((8, 1, NP // 8), _f32),
    )(pp)


def _sc_mesh():
    return plsc.VectorSubcoreMesh(core_axis_name="c", subcore_axis_name="s")


_SC_PARAMS = pltpu.CompilerParams(needs_layout_passes=False)


@functools.partial(
    pl.kernel,
    out_type=[
        jax.ShapeDtypeStruct((EP,), _f32),
        jax.ShapeDtypeStruct((NW * NP,), _f32),
    ],
    mesh=_sc_mesh(),
    compiler_params=_SC_PARAMS,
    scratch_types=[
        pltpu.VMEM((NP,), _f32),
        pltpu.VMEM((NP,), _f32),
        pltpu.VMEM((NP,), _f32),
        pltpu.VMEM((16,), _f32),
        pltpu.VMEM((PW,), _i32),
        pltpu.VMEM((PW,), _i32),
        pltpu.VMEM((PW,), _f32),
        pltpu.VMEM((PW,), _f32),
    ],
)
def _sc_pass1(s_hbm, d_hbm, ce_hbm, src_hbm, dst_hbm, c_hbm,
              p_hbm, sspart_hbm,
              s_t, d_t, ssum_t, c_t, src_t, dst_t, ce_t, p_t):
    cid = lax.axis_index("c")
    sid = lax.axis_index("s")
    wid = cid * NS + sid
    base = wid * PW
    pltpu.sync_copy(s_hbm, s_t)
    pltpu.sync_copy(d_hbm, d_t)
    pltpu.sync_copy(c_hbm, c_t)
    pltpu.sync_copy(src_hbm.at[pl.ds(base, PW)], src_t)
    pltpu.sync_copy(dst_hbm.at[pl.ds(base, PW)], dst_t)
    pltpu.sync_copy(ce_hbm.at[pl.ds(base, PW)], ce_t)
    cv = c_t[...]

    def zero(i, carry):
        ssum_t[pl.ds(i * 16, 16)] = jnp.zeros((16,), _f32)
        return carry
    lax.fori_loop(0, NP // 16, zero, 0)

    def grp(j, carry):
        sl = pl.ds(j * 16, 16)
        si = src_t[sl]
        di = dst_t[sl]
        a = plsc.load_gather(s_t, [si]) + plsc.load_gather(d_t, [di]) \
            + ce_t[sl]
        a = jnp.where(a >= 0, a, a * 0.2)
        pv = jnp.exp(a - cv)
        p_t[sl] = pv
        plsc.addupdate_scatter(ssum_t, [di], pv)
        return carry
    lax.fori_loop(0, PW // 16, grp, 0)
    pltpu.sync_copy(p_t, p_hbm.at[pl.ds(base, PW)])
    pltpu.sync_copy(ssum_t, sspart_hbm.at[pl.ds(wid * NP, NP)])


@functools.partial(
    pl.kernel,
    out_type=[
        jax.ShapeDtypeStruct((EP,), _f32),
        jax.ShapeDtypeStruct((NC, NP, H), _f32),
    ],
    mesh=_sc_mesh(),
    compiler_params=_SC_PARAMS,
    scratch_types=[
        pltpu.VMEM((NP,), _f32),
        pltpu.VMEM((SUP,), _i32),
        pltpu.VMEM((SUP,), _i32),
        pltpu.VMEM((SUP,), _f32),
        pltpu.VMEM((SUP,), _f32),
        pltpu.VMEM((K2,), _i32),
        pltpu.VMEM((K2,), _i32),
        pltpu.VMEM((K2, H), _f32),
        pltpu.VMEM((K2, H), _f32),
        pltpu.VMEM_SHARED((NP, H), _f32),
        pltpu.SemaphoreType.DMA,
        pltpu.SemaphoreType.DMA,
        pltpu.SemaphoreType.DMA,
        pltpu.SemaphoreType.DMA,
    ],
)
def _sc_pass2(ssum_hbm, p_hbm, src_hbm, dst_hbm, h_hbm, z_hbm,
              w_hbm, outp_hbm,
              ssum_t, src_t, dst_t, p_t, w_t, dcA, dcB, rowsA, rowsB,
              acc_sh, sgA, sgB, ssA, ssB):
    cid = lax.axis_index("c")
    sid = lax.axis_index("s")
    wid = cid * NS + sid
    stripe = NP // NS
    CH = SUP // K2

    pltpu.sync_copy(ssum_hbm, ssum_t)
    pltpu.sync_copy(z_hbm, acc_sh.at[pl.ds(sid * stripe, stripe)])
    plsc.subcore_barrier()

    rbufs = (rowsA, rowsB)
    dbufs = (dcA, dcB)
    gsems = (sgA, sgB)
    ssems = (ssA, ssB)

    def do_super(s, carry):
        sb = wid * PW + s * SUP
        pltpu.sync_copy(src_hbm.at[pl.ds(sb, SUP)], src_t)
        pltpu.sync_copy(dst_hbm.at[pl.ds(sb, SUP)], dst_t)
        pltpu.sync_copy(p_hbm.at[pl.ds(sb, SUP)], p_t)
        scat = [None, None]
        gat = [pltpu.async_copy(h_hbm.at[src_t.at[pl.ds(0, K2)]],
                                rowsA, sgA), None]
        for c in range(CH):
            cur = c % 2
            nxt = 1 - cur
            if c >= 1:
                scat[nxt].wait()
            if c + 1 < CH:
                gat[nxt] = pltpu.async_copy(
                    h_hbm.at[src_t.at[pl.ds((c + 1) * K2, K2)]],
                    rbufs[nxt], gsems[nxt])
            gat[cur].wait()
            co = c * K2

            def wgrp(j, carry2, _co=co, _db=dbufs[cur]):
                sl16 = pl.ds(_co + j * 16, 16)
                di = dst_t[sl16]
                sv = plsc.load_gather(ssum_t, [di])
                w_t[sl16] = p_t[sl16] / (sv + 1e-16)
                _db[pl.ds(j * 16, 16)] = di
                return carry2
            lax.fori_loop(0, K2 // 16, wgrp, 0)

            def rowf(r, carry2, _co=co, _rb=rbufs[cur]):
                wb = plsc.load_gather(w_t, [jnp.full((16,), _co, _i32) + r])
                for cg in range(H // 16):
                    cs = pl.ds(cg * 16, 16)
                    _rb[r, cs] = _rb[r, cs] * wb
                return carry2
            lax.fori_loop(0, K2, rowf, 0)
            scat[cur] = pltpu.async_copy(rbufs[cur], acc_sh.at[dbufs[cur]],
                                         ssems[cur], add=True)
        scat[(CH - 1) % 2].wait()
        pltpu.sync_copy(w_t, w_hbm.at[pl.ds(sb, SUP)])
        return carry
    lax.fori_loop(0, PW // SUP, do_super, 0)

    plsc.subcore_barrier()
    for i in range(stripe // 64):
        row0 = sid * stripe + i * 64
        pltpu.sync_copy(acc_sh.at[pl.ds(row0, 64)], rowsA.at[pl.ds(0, 64)])
        pltpu.sync_copy(rowsA.at[pl.ds(0, 64)],
                        outp_hbm.at[cid, pl.ds(row0, 64)])


@functools.partial(
    pl.kernel,
    out_type=jax.ShapeDtypeStruct((NW, G, H), _f32),
    mesh=_sc_mesh(),
    compiler_params=_SC_PARAMS,
    scratch_types=[
        pltpu.VMEM((G, H), _f32),
        pltpu.VMEM((64, H), _f32),
        pltpu.VMEM((NP // NW,), _i32),
    ],
)
def _sc_pool(hr_hbm, bat_hbm, z_hbm, pool_hbm, tbl_t, rows_t, bat_t):
    cid = lax.axis_index("c")
    sid = lax.axis_index("s")
    wid = cid * NS + sid
    rpw = NP // NW
    pltpu.sync_copy(z_hbm.at[pl.ds(0, G)], tbl_t)
    pltpu.sync_copy(bat_hbm.at[pl.ds(wid * rpw, rpw)], bat_t)
    colio = lax.iota(_i32, 16)

    def chunk(i, carry):
        pltpu.sync_copy(hr_hbm.at[pl.ds(wid * rpw + i * 64, 64)], rows_t)

        def row(r, carry2):
            gv = plsc.load_gather(bat_t, [jnp.full((16,), i * 64, _i32) + r])
            ri = jnp.full((16,), r, _i32)
            for c in range(H // 16):
                ci = colio + (c * 16)
                v = plsc.load_gather(rows_t, [ri, ci])
                cur = plsc.load_gather(tbl_t, [gv, ci])
                plsc.store_scatter(tbl_t, [gv, ci], jnp.maximum(cur, v))
            return carry2
        lax.fori_loop(0, 64, row, 0)
        return carry
    lax.fori_loop(0, rpw // 64, chunk, 0)
    pltpu.sync_copy(tbl_t, pool_hbm.at[wid])


def kernel(x, edge_index, edge_attr, batch, W1, as1, ad1, ae1, We1, b1,
           W2, as2, ad2, ae2, We2, b2, ln_g, ln_b, fc_w, fc_b):
    loop = jnp.arange(N, dtype=_i32)
    padi = jnp.zeros((EP - ET,), _i32)
    src = jnp.concatenate([edge_index[0].astype(_i32), loop, padi])
    dst = jnp.concatenate([edge_index[1].astype(_i32), loop, padi])

    ce1e, ce2e, est = _edge_pre(edge_attr, We1, ae1.reshape(H, 1),
                                We2, ae2.reshape(H, 1))
    mean1 = jnp.sum(est[:, 0, 0]) / E
    mean2 = jnp.sum(est[:, 0, 2]) / E
    maxce1 = jnp.maximum(jnp.max(est[:, 0, 1]), mean1)
    maxce2 = jnp.maximum(jnp.max(est[:, 0, 3]), mean2)
    padf = jnp.full((EP - ET,), -1e30, _f32)
    ce1 = jnp.concatenate([ce1e.reshape(-1), jnp.full((N,), mean1, _f32), padf])
    ce2 = jnp.concatenate([ce2e.reshape(-1), jnp.full((N,), mean2, _f32), padf])

    zrows = jnp.zeros((NP // NS, H), _f32)
    b1r = b1.reshape(1, H)
    b2r = b2.reshape(1, H)
    gr = ln_g.reshape(1, H)
    blr = ln_b.reshape(1, H)

    def _padn(v):
        return jnp.concatenate([v.reshape(-1), jnp.zeros((NP - N,), _f32)])

    h1, s1, d1, nst1 = _node_pre(x, W1, as1.reshape(H, 1), ad1.reshape(H, 1))
    c1 = jnp.maximum(jnp.max(nst1[:, 0, 0]) + jnp.max(nst1[:, 0, 1]) + maxce1,
                     0.0)
    p1, sspart1 = _sc_pass1(_padn(s1), _padn(d1), ce1, src, dst,
                            jnp.full((16,), c1, _f32))
    ssum1 = _ssum_reduce(sspart1.reshape(NW, NP)).reshape(NP)
    w1, outp1 = _sc_pass2(ssum1, p1, src, dst, h1, zrows)

    h2, s2, d2, nst2 = _post1(outp1[0, :N], outp1[1, :N], b1r, gr, blr,
                              W2, as2.reshape(H, 1), ad2.reshape(H, 1))
    c2 = jnp.maximum(jnp.max(nst2[:, 0, 0]) + jnp.max(nst2[:, 0, 1]) + maxce2,
                     0.0)
    p2, sspart2 = _sc_pass1(_padn(s2), _padn(d2), ce2, src, dst,
                            jnp.full((16,), c2, _f32))
    ssum2 = _ssum_reduce(sspart2.reshape(NW, NP)).reshape(NP)
    w2, outp2 = _sc_pass2(ssum2, p2, src, dst, h2, zrows)

    hrel = _post2(outp2[0, :N], outp2[1, :N], b2r, gr, blr)
    hrelp = jnp.concatenate([hrel, jnp.zeros((NP - N, H), _f32)], axis=0)
    batp = jnp.concatenate([batch.astype(_i32), jnp.zeros((NP - N,), _i32)])
    pool = _sc_pool(hrelp, batp, zrows)
    out = _pool_final(pool, fc_w, fc_b.reshape(1, 1))
    return (out.reshape(-1), w1[:ET], w2[:ET])


# speedup vs baseline: 21.2599x; 1.0615x over previous
import functools

import jax
import jax.numpy as jnp
from jax import lax
from jax.experimental import pallas as pl
from jax.experimental.pallas import tpu as pltpu
from jax.experimental.pallas import tpu_sc as plsc

N = 10000
E = 320000
D = 128
H = 128
DE = 16
G = 64
ET = E + N
NC = 2
NS = 16
NW = NC * NS
PW = 10368
EP = NW * PW
NP = 10240
K1 = 192
K2 = 96
SUP = 36 * K2
EB = 4000
XB = 1000

_f32 = jnp.float32
_i32 = jnp.int32


def _edge_pre_body(ea_ref, we1_ref, ae1_ref, we2_ref, ae2_ref,
                   ce1_ref, ce2_ref, st_ref):
    ea = ea_ref[...]
    ve1 = jnp.dot(we1_ref[...], ae1_ref[...], preferred_element_type=_f32)
    ve2 = jnp.dot(we2_ref[...], ae2_ref[...], preferred_element_type=_f32)
    ce1 = jnp.dot(ea, ve1, preferred_element_type=_f32)
    ce2 = jnp.dot(ea, ve2, preferred_element_type=_f32)
    ce1_ref[...] = ce1
    ce2_ref[...] = ce2
    st_ref[...] = jnp.concatenate(
        [jnp.sum(ce1).reshape(1, 1, 1), jnp.max(ce1).reshape(1, 1, 1),
         jnp.sum(ce2).reshape(1, 1, 1), jnp.max(ce2).reshape(1, 1, 1)],
        axis=2)


def _edge_pre(ea, we1, ae1, we2, ae2):
    nb = E // EB
    return pl.pallas_call(
        _edge_pre_body,
        grid=(nb,),
        in_specs=[
            pl.BlockSpec((EB, DE), lambda i: (i, 0)),
            pl.BlockSpec((DE, H), lambda i: (0, 0)),
            pl.BlockSpec((H, 1), lambda i: (0, 0)),
            pl.BlockSpec((DE, H), lambda i: (0, 0)),
            pl.BlockSpec((H, 1), lambda i: (0, 0)),
        ],
        out_specs=[
            pl.BlockSpec((EB, 1), lambda i: (i, 0)),
            pl.BlockSpec((EB, 1), lambda i: (i, 0)),
            pl.BlockSpec((1, 1, 4), lambda i: (i, 0, 0)),
        ],
        out_shape=[
            jax.ShapeDtypeStruct((E, 1), _f32),
            jax.ShapeDtypeStruct((E, 1), _f32),
            jax.ShapeDtypeStruct((nb, 1, 4), _f32),
        ],
    )(ea, we1, ae1, we2, ae2)


def _node_pre_body(x_ref, w_ref, as_ref, ad_ref, h_ref, s_ref, d_ref, st_ref):
    h = jnp.dot(x_ref[...], w_ref[...], preferred_element_type=_f32)
    s = jnp.dot(h, as_ref[...], preferred_element_type=_f32)
    d = jnp.dot(h, ad_ref[...], preferred_element_type=_f32)
    h_ref[...] = h
    s_ref[...] = s
    d_ref[...] = d
    st_ref[...] = jnp.concatenate(
        [jnp.max(s).reshape(1, 1, 1), jnp.max(d).reshape(1, 1, 1)], axis=2)


def _node_pre(x, w, a_s, a_d):
    nb = N // XB
    return pl.pallas_call(
        _node_pre_body,
        grid=(nb,),
        in_specs=[
            pl.BlockSpec((XB, D), lambda i: (i, 0)),
            pl.BlockSpec((D, H), lambda i: (0, 0)),
            pl.BlockSpec((H, 1), lambda i: (0, 0)),
            pl.BlockSpec((H, 1), lambda i: (0, 0)),
        ],
        out_specs=[
            pl.BlockSpec((XB, H), lambda i: (i, 0)),
            pl.BlockSpec((XB, 1), lambda i: (i, 0)),
            pl.BlockSpec((XB, 1), lambda i: (i, 0)),
            pl.BlockSpec((1, 1, 2), lambda i: (i, 0, 0)),
        ],
        out_shape=[
            jax.ShapeDtypeStruct((N, H), _f32),
            jax.ShapeDtypeStruct((N, 1), _f32),
            jax.ShapeDtypeStruct((N, 1), _f32),
            jax.ShapeDtypeStruct((nb, 1, 2), _f32),
        ],
    )(x, w, a_s, a_d)


def _ln_lrelu(o, g_ref, bl_ref):
    mu = jnp.mean(o, axis=-1, keepdims=True)
    c = o - mu
    var = jnp.mean(c * c, axis=-1, keepdims=True)
    t = c * lax.rsqrt(var + 1e-5) * g_ref[...] + bl_ref[...]
    return jnp.where(t >= 0, t, t * 0.1)


def _post1_body(o0_ref, o1_ref, b_ref, g_ref, bl_ref, w_ref, as_ref, ad_ref,
                h_ref, s_ref, d_ref, st_ref):
    o = o0_ref[...] + o1_ref[...] + b_ref[...]
    t = _ln_lrelu(o, g_ref, bl_ref)
    h = jnp.dot(t, w_ref[...], preferred_element_type=_f32)
    s = jnp.dot(h, as_ref[...], preferred_element_type=_f32)
    d = jnp.dot(h, ad_ref[...], preferred_element_type=_f32)
    h_ref[...] = h
    s_ref[...] = s
    d_ref[...] = d
    st_ref[...] = jnp.concatenate(
        [jnp.max(s).reshape(1, 1, 1), jnp.max(d).reshape(1, 1, 1)], axis=2)


def _post1(o0, o1, b, g, bl, w, a_s, a_d):
    nb = N // XB
    return pl.pallas_call(
        _post1_body,
        grid=(nb,),
        in_specs=[
            pl.BlockSpec((XB, H), lambda i: (i, 0)),
            pl.BlockSpec((XB, H), lambda i: (i, 0)),
            pl.BlockSpec((1, H), lambda i: (0, 0)),
            pl.BlockSpec((1, H), lambda i: (0, 0)),
            pl.BlockSpec((1, H), lambda i: (0, 0)),
            pl.BlockSpec((H, H), lambda i: (0, 0)),
            pl.BlockSpec((H, 1), lambda i: (0, 0)),
            pl.BlockSpec((H, 1), lambda i: (0, 0)),
        ],
        out_specs=[
            pl.BlockSpec((XB, H), lambda i: (i, 0)),
            pl.BlockSpec((XB, 1), lambda i: (i, 0)),
            pl.BlockSpec((XB, 1), lambda i: (i, 0)),
            pl.BlockSpec((1, 1, 2), lambda i: (i, 0, 0)),
        ],
        out_shape=[
            jax.ShapeDtypeStruct((N, H), _f32),
            jax.ShapeDtypeStruct((N, 1), _f32),
            jax.ShapeDtypeStruct((N, 1), _f32),
            jax.ShapeDtypeStruct((nb, 1, 2), _f32),
        ],
    )(o0, o1, b, g, bl, w, a_s, a_d)


def _post2_body(o0_ref, o1_ref, b_ref, g_ref, bl_ref, hr_ref):
    o = o0_ref[...] + o1_ref[...] + b_ref[...]
    t = _ln_lrelu(o, g_ref, bl_ref)
    hr_ref[...] = jnp.maximum(t, 0.0)


def _post2(o0, o1, b, g, bl):
    nb = N // XB
    return pl.pallas_call(
        _post2_body,
        grid=(nb,),
        in_specs=[
            pl.BlockSpec((XB, H), lambda i: (i, 0)),
            pl.BlockSpec((XB, H), lambda i: (i, 0)),
            pl.BlockSpec((1, H), lambda i: (0, 0)),
            pl.BlockSpec((1, H), lambda i: (0, 0)),
            pl.BlockSpec((1, H), lambda i: (0, 0)),
        ],
        out_specs=pl.BlockSpec((XB, H), lambda i: (i, 0)),
        out_shape=jax.ShapeDtypeStruct((N, H), _f32),
    )(o0, o1, b, g, bl)


def _pool_final_body(pp_ref, fw_ref, fb_ref, o_ref):
    m = jnp.max(pp_ref[...], axis=0)
    o_ref[...] = jnp.dot(m, fw_ref[...], preferred_element_type=_f32) \
        + fb_ref[...]


def _pool_final(pp, fw, fb):
    return pl.pallas_call(
        _pool_final_body,
        grid=(G // 8,),
        in_specs=[
            pl.BlockSpec((NW, 8, H), lambda i: (0, i, 0)),
            pl.BlockSpec((H, 1), lambda i: (0, 0)),
            pl.BlockSpec((1, 1), lambda i: (0, 0)),
        ],
        out_specs=pl.BlockSpec((8, 1), lambda i: (i, 0)),
        out_shape=jax.ShapeDtypeStruct((G, 1), _f32),
    )(pp, fw, fb)


def _ssum_reduce_body(pp_ref, o_ref):
    o_ref[...] = jnp.sum(pp_ref[...], axis=0, keepdims=True)[None]


def _ssum_reduce(pp):
    return pl.pallas_call(
        _ssum_reduce_body,
        grid=(8,),
        in_specs=[pl.BlockSpec((NW, NP // 8), lambda i: (0, i))],
        out_specs=pl.BlockSpec((1, 1, NP // 8), lambda i: (i, 0, 0)),
        out_shape=jax.ShapeDtypeStruct((8, 1, NP // 8), _f32),
    )(pp)


def _sc_mesh():
    return plsc.VectorSubcoreMesh(core_axis_name="c", subcore_axis_name="s")


_SC_PARAMS = pltpu.CompilerParams(needs_layout_passes=False)


@functools.partial(
    pl.kernel,
    out_type=[
        jax.ShapeDtypeStruct((EP,), _f32),
        jax.ShapeDtypeStruct((NW * NP,), _f32),
    ],
    mesh=_sc_mesh(),
    compiler_params=_SC_PARAMS,
    scratch_types=[
        pltpu.VMEM((NP,), _f32),
        pltpu.VMEM((NP,), _f32),
        pltpu.VMEM((NP,), _f32),
        pltpu.VMEM((16,), _f32),
        pltpu.VMEM((PW,), _i32),
        pltpu.VMEM((PW,), _i32),
        pltpu.VMEM((PW,), _f32),
        pltpu.VMEM((PW,), _f32),
    ],
)
def _sc_pass1(s_hbm, d_hbm, ce_hbm, src_hbm, dst_hbm, c_hbm,
              p_hbm, sspart_hbm,
              s_t, d_t, ssum_t, c_t, src_t, dst_t, ce_t, p_t):
    cid = lax.axis_index("c")
    sid = lax.axis_index("s")
    wid = cid * NS + sid
    base = wid * PW
    pltpu.sync_copy(s_hbm, s_t)
    pltpu.sync_copy(d_hbm, d_t)
    pltpu.sync_copy(c_hbm, c_t)
    pltpu.sync_copy(src_hbm.at[pl.ds(base, PW)], src_t)
    pltpu.sync_copy(dst_hbm.at[pl.ds(base, PW)], dst_t)
    pltpu.sync_copy(ce_hbm.at[pl.ds(base, PW)], ce_t)
    cv = c_t[...]

    def zero(i, carry):
        ssum_t[pl.ds(i * 16, 16)] = jnp.zeros((16,), _f32)
        return carry
    lax.fori_loop(0, NP // 16, zero, 0)

    def grp(j, carry):
        sl = pl.ds(j * 16, 16)
        si = src_t[sl]
        di = dst_t[sl]
        a = plsc.load_gather(s_t, [si]) + plsc.load_gather(d_t, [di]) \
            + ce_t[sl]
        a = jnp.where(a >= 0, a, a * 0.2)
        pv = jnp.exp(a - cv)
        p_t[sl] = pv
        plsc.addupdate_scatter(ssum_t, [di], pv)
        return carry
    lax.fori_loop(0, PW // 16, grp, 0)
    pltpu.sync_copy(p_t, p_hbm.at[pl.ds(base, PW)])
    pltpu.sync_copy(ssum_t, sspart_hbm.at[pl.ds(wid * NP, NP)])


@functools.partial(
    pl.kernel,
    out_type=[
        jax.ShapeDtypeStruct((EP,), _f32),
        jax.ShapeDtypeStruct((NC, NP, H), _f32),
    ],
    mesh=_sc_mesh(),
    compiler_params=_SC_PARAMS,
    scratch_types=[
        pltpu.VMEM((NP,), _f32),
        pltpu.VMEM((SUP,), _i32),
        pltpu.VMEM((SUP,), _i32),
        pltpu.VMEM((SUP,), _f32),
        pltpu.VMEM((SUP,), _f32),
        pltpu.VMEM((K2,), _i32),
        pltpu.VMEM((K2,), _i32),
        pltpu.VMEM((K2, H), _f32),
        pltpu.VMEM((K2, H), _f32),
        pltpu.VMEM_SHARED((NP, H), _f32),
        pltpu.SemaphoreType.DMA,
        pltpu.SemaphoreType.DMA,
        pltpu.SemaphoreType.DMA,
        pltpu.SemaphoreType.DMA,
    ],
)
def _sc_pass2(ssum_hbm, p_hbm, src_hbm, dst_hbm, h_hbm, z_hbm,
              w_hbm, outp_hbm,
              ssum_t, src_t, dst_t, p_t, w_t, dcA, dcB, rowsA, rowsB,
              acc_sh, sgA, sgB, ssA, ssB):
    cid = lax.axis_index("c")
    sid = lax.axis_index("s")
    wid = cid * NS + sid
    stripe = NP // NS
    CH = SUP // K2

    pltpu.sync_copy(ssum_hbm, ssum_t)
    pltpu.sync_copy(z_hbm, acc_sh.at[pl.ds(sid * stripe, stripe)])
    plsc.subcore_barrier()

    rbufs = (rowsA, rowsB)
    dbufs = (dcA, dcB)
    gsems = (sgA, sgB)
    ssems = (ssA, ssB)

    def do_super(s, carry):
        sb = wid * PW + s * SUP
        pltpu.sync_copy(src_hbm.at[pl.ds(sb, SUP)], src_t)
        pltpu.sync_copy(dst_hbm.at[pl.ds(sb, SUP)], dst_t)
        pltpu.sync_copy(p_hbm.at[pl.ds(sb, SUP)], p_t)
        scat = [None, None]
        gat = [pltpu.async_copy(h_hbm.at[src_t.at[pl.ds(0, K2)]],
                                rowsA, sgA), None]
        for c in range(CH):
            cur = c % 2
            nxt = 1 - cur
            if c >= 1:
                scat[nxt].wait()
            if c + 1 < CH:
                gat[nxt] = pltpu.async_copy(
                    h_hbm.at[src_t.at[pl.ds((c + 1) * K2, K2)]],
                    rbufs[nxt], gsems[nxt])
            gat[cur].wait()
            co = c * K2

            def wgrp(j, carry2, _co=co, _db=dbufs[cur]):
                sl16 = pl.ds(_co + j * 16, 16)
                di = dst_t[sl16]
                sv = plsc.load_gather(ssum_t, [di])
                w_t[sl16] = p_t[sl16] / (sv + 1e-16)
                _db[pl.ds(j * 16, 16)] = di
                return carry2
            lax.fori_loop(0, K2 // 16, wgrp, 0)

            def rowf(r2, carry2, _co=co, _rb=rbufs[cur]):
                r = r2 * 2
                wb0 = plsc.load_gather(w_t, [jnp.full((16,), _co, _i32) + r])
                wb1 = plsc.load_gather(
                    w_t, [jnp.full((16,), _co + 1, _i32) + r])
                for cg in range(H // 16):
                    cs = pl.ds(cg * 16, 16)
                    _rb[r, cs] = _rb[r, cs] * wb0
                    _rb[r + 1, cs] = _rb[r + 1, cs] * wb1
                return carry2
            lax.fori_loop(0, K2 // 2, rowf, 0)
            scat[cur] = pltpu.async_copy(rbufs[cur], acc_sh.at[dbufs[cur]],
                                         ssems[cur], add=True)
        scat[(CH - 1) % 2].wait()
        pltpu.sync_copy(w_t, w_hbm.at[pl.ds(sb, SUP)])
        return carry
    lax.fori_loop(0, PW // SUP, do_super, 0)

    plsc.subcore_barrier()
    for i in range(stripe // 64):
        row0 = sid * stripe + i * 64
        pltpu.sync_copy(acc_sh.at[pl.ds(row0, 64)], rowsA.at[pl.ds(0, 64)])
        pltpu.sync_copy(rowsA.at[pl.ds(0, 64)],
                        outp_hbm.at[cid, pl.ds(row0, 64)])


@functools.partial(
    pl.kernel,
    out_type=jax.ShapeDtypeStruct((NW, G, H), _f32),
    mesh=_sc_mesh(),
    compiler_params=_SC_PARAMS,
    scratch_types=[
        pltpu.VMEM((G, H), _f32),
        pltpu.VMEM((64, H), _f32),
        pltpu.VMEM((NP // NW,), _i32),
    ],
)
def _sc_pool(hr_hbm, bat_hbm, z_hbm, pool_hbm, tbl_t, rows_t, bat_t):
    cid = lax.axis_index("c")
    sid = lax.axis_index("s")
    wid = cid * NS + sid
    rpw = NP // NW
    pltpu.sync_copy(z_hbm.at[pl.ds(0, G)], tbl_t)
    pltpu.sync_copy(bat_hbm.at[pl.ds(wid * rpw, rpw)], bat_t)
    colio = lax.iota(_i32, 16)

    def chunk(i, carry):
        pltpu.sync_copy(hr_hbm.at[pl.ds(wid * rpw + i * 64, 64)], rows_t)

        def row(r, carry2):
            gv = plsc.load_gather(bat_t, [jnp.full((16,), i * 64, _i32) + r])
            ri = jnp.full((16,), r, _i32)
            for c in range(H // 16):
                ci = colio + (c * 16)
                v = plsc.load_gather(rows_t, [ri, ci])
                cur = plsc.load_gather(tbl_t, [gv, ci])
                plsc.store_scatter(tbl_t, [gv, ci], jnp.maximum(cur, v))
            return carry2
        lax.fori_loop(0, 64, row, 0)
        return carry
    lax.fori_loop(0, rpw // 64, chunk, 0)
    pltpu.sync_copy(tbl_t, pool_hbm.at[wid])


def kernel(x, edge_index, edge_attr, batch, W1, as1, ad1, ae1, We1, b1,
           W2, as2, ad2, ae2, We2, b2, ln_g, ln_b, fc_w, fc_b):
    loop = jnp.arange(N, dtype=_i32)
    padi = jnp.zeros((EP - ET,), _i32)
    src = jnp.concatenate([edge_index[0].astype(_i32), loop, padi])
    dst = jnp.concatenate([edge_index[1].astype(_i32), loop, padi])

    ce1e, ce2e, est = _edge_pre(edge_attr, We1, ae1.reshape(H, 1),
                                We2, ae2.reshape(H, 1))
    mean1 = jnp.sum(est[:, 0, 0]) / E
    mean2 = jnp.sum(est[:, 0, 2]) / E
    maxce1 = jnp.maximum(jnp.max(est[:, 0, 1]), mean1)
    maxce2 = jnp.maximum(jnp.max(est[:, 0, 3]), mean2)
    padf = jnp.full((EP - ET,), -1e30, _f32)
    ce1 = jnp.concatenate([ce1e.reshape(-1), jnp.full((N,), mean1, _f32), padf])
    ce2 = jnp.concatenate([ce2e.reshape(-1), jnp.full((N,), mean2, _f32), padf])

    zrows = jnp.zeros((NP // NS, H), _f32)
    b1r = b1.reshape(1, H)
    b2r = b2.reshape(1, H)
    gr = ln_g.reshape(1, H)
    blr = ln_b.reshape(1, H)

    def _padn(v):
        return jnp.concatenate([v.reshape(-1), jnp.zeros((NP - N,), _f32)])

    h1, s1, d1, nst1 = _node_pre(x, W1, as1.reshape(H, 1), ad1.reshape(H, 1))
    c1 = jnp.maximum(jnp.max(nst1[:, 0, 0]) + jnp.max(nst1[:, 0, 1]) + maxce1,
                     0.0)
    p1, sspart1 = _sc_pass1(_padn(s1), _padn(d1), ce1, src, dst,
                            jnp.full((16,), c1, _f32))
    ssum1 = _ssum_reduce(sspart1.reshape(NW, NP)).reshape(NP)
    w1, outp1 = _sc_pass2(ssum1, p1, src, dst, h1, zrows)

    h2, s2, d2, nst2 = _post1(outp1[0, :N], outp1[1, :N], b1r, gr, blr,
                              W2, as2.reshape(H, 1), ad2.reshape(H, 1))
    c2 = jnp.maximum(jnp.max(nst2[:, 0, 0]) + jnp.max(nst2[:, 0, 1]) + maxce2,
                     0.0)
    p2, sspart2 = _sc_pass1(_padn(s2), _padn(d2), ce2, src, dst,
                            jnp.full((16,), c2, _f32))
    ssum2 = _ssum_reduce(sspart2.reshape(NW, NP)).reshape(NP)
    w2, outp2 = _sc_pass2(ssum2, p2, src, dst, h2, zrows)

    hrel = _post2(outp2[0, :N], outp2[1, :N], b2r, gr, blr)
    hrelp = jnp.concatenate([hrel, jnp.zeros((NP - N, H), _f32)], axis=0)
    batp = jnp.concatenate([batch.astype(_i32), jnp.zeros((NP - N,), _i32)])
    pool = _sc_pool(hrelp, batp, zrows)
    out = _pool_final(pool, fc_w, fc_b.reshape(1, 1))
    return (out.reshape(-1), w1[:ET], w2[:ET])


# speedup vs baseline: 25.6563x; 1.2068x over previous
import functools

import jax
import jax.numpy as jnp
from jax import lax
from jax.experimental import pallas as pl
from jax.experimental.pallas import tpu as pltpu
from jax.experimental.pallas import tpu_sc as plsc

N = 10000
E = 320000
D = 128
H = 128
DE = 16
G = 64
ET = E + N
NC = 2
NS = 16
NW = NC * NS
PW = 10368
EP = NW * PW
NP = 10240
K1 = 192
K2 = 96
SUP = 36 * K2
EB = 4000
XB = 1000

_f32 = jnp.float32
_i32 = jnp.int32


def _edge_pre_body(eap_ref, wt1_ref, ae1_ref, wt2_ref, ae2_ref,
                   ce1_ref, ce2_ref, st_ref):
    eap = eap_ref[...]
    ve1 = jnp.dot(ae1_ref[...], wt1_ref[...], preferred_element_type=_f32)
    ve2 = jnp.dot(ae2_ref[...], wt2_ref[...], preferred_element_type=_f32)
    v1 = jnp.concatenate([ve1] * 8, axis=1)
    v2 = jnp.concatenate([ve2] * 8, axis=1)
    jj = lax.broadcasted_iota(_i32, (128, 8), 0) // DE
    kk = lax.broadcasted_iota(_i32, (128, 8), 1)
    sel = jnp.where(jj == kk, 1.0, 0.0).astype(_f32)
    ce1 = jnp.dot(eap * v1, sel, preferred_element_type=_f32)
    ce2 = jnp.dot(eap * v2, sel, preferred_element_type=_f32)
    ce1_ref[...] = ce1
    ce2_ref[...] = ce2
    st_ref[...] = jnp.concatenate(
        [jnp.sum(ce1).reshape(1, 1, 1), jnp.max(ce1).reshape(1, 1, 1),
         jnp.sum(ce2).reshape(1, 1, 1), jnp.max(ce2).reshape(1, 1, 1)],
        axis=2)


def _edge_pre(eap, wt1, ae1, wt2, ae2):
    nb = 10
    eb8 = E // 8 // nb
    return pl.pallas_call(
        _edge_pre_body,
        grid=(nb,),
        in_specs=[
            pl.BlockSpec((eb8, 128), lambda i: (i, 0)),
            pl.BlockSpec((H, DE), lambda i: (0, 0)),
            pl.BlockSpec((1, H), lambda i: (0, 0)),
            pl.BlockSpec((H, DE), lambda i: (0, 0)),
            pl.BlockSpec((1, H), lambda i: (0, 0)),
        ],
        out_specs=[
            pl.BlockSpec((eb8, 8), lambda i: (i, 0)),
            pl.BlockSpec((eb8, 8), lambda i: (i, 0)),
            pl.BlockSpec((1, 1, 4), lambda i: (i, 0, 0)),
        ],
        out_shape=[
            jax.ShapeDtypeStruct((E // 8, 8), _f32),
            jax.ShapeDtypeStruct((E // 8, 8), _f32),
            jax.ShapeDtypeStruct((nb, 1, 4), _f32),
        ],
    )(eap, wt1, ae1, wt2, ae2)


def _node_pre_body(x_ref, w_ref, as_ref, ad_ref, h_ref, s_ref, d_ref, st_ref):
    h = jnp.dot(x_ref[...], w_ref[...], preferred_element_type=_f32)
    s = jnp.dot(h, as_ref[...], preferred_element_type=_f32)
    d = jnp.dot(h, ad_ref[...], preferred_element_type=_f32)
    h_ref[...] = h
    s_ref[...] = s
    d_ref[...] = d
    st_ref[...] = jnp.concatenate(
        [jnp.max(s).reshape(1, 1, 1), jnp.max(d).reshape(1, 1, 1)], axis=2)


def _node_pre(x, w, a_s, a_d):
    nb = N // XB
    return pl.pallas_call(
        _node_pre_body,
        grid=(nb,),
        in_specs=[
            pl.BlockSpec((XB, D), lambda i: (i, 0)),
            pl.BlockSpec((D, H), lambda i: (0, 0)),
            pl.BlockSpec((H, 1), lambda i: (0, 0)),
            pl.BlockSpec((H, 1), lambda i: (0, 0)),
        ],
        out_specs=[
            pl.BlockSpec((XB, H), lambda i: (i, 0)),
            pl.BlockSpec((XB, 1), lambda i: (i, 0)),
            pl.BlockSpec((XB, 1), lambda i: (i, 0)),
            pl.BlockSpec((1, 1, 2), lambda i: (i, 0, 0)),
        ],
        out_shape=[
            jax.ShapeDtypeStruct((N, H), _f32),
            jax.ShapeDtypeStruct((N, 1), _f32),
            jax.ShapeDtypeStruct((N, 1), _f32),
            jax.ShapeDtypeStruct((nb, 1, 2), _f32),
        ],
    )(x, w, a_s, a_d)


def _ln_lrelu(o, g_ref, bl_ref):
    mu = jnp.mean(o, axis=-1, keepdims=True)
    c = o - mu
    var = jnp.mean(c * c, axis=-1, keepdims=True)
    t = c * lax.rsqrt(var + 1e-5) * g_ref[...] + bl_ref[...]
    return jnp.where(t >= 0, t, t * 0.1)


def _post1_body(o0_ref, o1_ref, b_ref, g_ref, bl_ref, w_ref, as_ref, ad_ref,
                h_ref, s_ref, d_ref, st_ref):
    o = o0_ref[...] + o1_ref[...] + b_ref[...]
    t = _ln_lrelu(o, g_ref, bl_ref)
    h = jnp.dot(t, w_ref[...], preferred_element_type=_f32)
    s = jnp.dot(h, as_ref[...], preferred_element_type=_f32)
    d = jnp.dot(h, ad_ref[...], preferred_element_type=_f32)
    h_ref[...] = h
    s_ref[...] = s
    d_ref[...] = d
    st_ref[...] = jnp.concatenate(
        [jnp.max(s).reshape(1, 1, 1), jnp.max(d).reshape(1, 1, 1)], axis=2)


def _post1(o0, o1, b, g, bl, w, a_s, a_d):
    nb = N // XB
    return pl.pallas_call(
        _post1_body,
        grid=(nb,),
        in_specs=[
            pl.BlockSpec((XB, H), lambda i: (i, 0)),
            pl.BlockSpec((XB, H), lambda i: (i, 0)),
            pl.BlockSpec((1, H), lambda i: (0, 0)),
            pl.BlockSpec((1, H), lambda i: (0, 0)),
            pl.BlockSpec((1, H), lambda i: (0, 0)),
            pl.BlockSpec((H, H), lambda i: (0, 0)),
            pl.BlockSpec((H, 1), lambda i: (0, 0)),
            pl.BlockSpec((H, 1), lambda i: (0, 0)),
        ],
        out_specs=[
            pl.BlockSpec((XB, H), lambda i: (i, 0)),
            pl.BlockSpec((XB, 1), lambda i: (i, 0)),
            pl.BlockSpec((XB, 1), lambda i: (i, 0)),
            pl.BlockSpec((1, 1, 2), lambda i: (i, 0, 0)),
        ],
        out_shape=[
            jax.ShapeDtypeStruct((N, H), _f32),
            jax.ShapeDtypeStruct((N, 1), _f32),
            jax.ShapeDtypeStruct((N, 1), _f32),
            jax.ShapeDtypeStruct((nb, 1, 2), _f32),
        ],
    )(o0, o1, b, g, bl, w, a_s, a_d)


def _post2_body(o0_ref, o1_ref, b_ref, g_ref, bl_ref, hr_ref):
    o = o0_ref[...] + o1_ref[...] + b_ref[...]
    t = _ln_lrelu(o, g_ref, bl_ref)
    hr_ref[...] = jnp.maximum(t, 0.0)


def _post2(o0, o1, b, g, bl):
    nb = N // XB
    return pl.pallas_call(
        _post2_body,
        grid=(nb,),
        in_specs=[
            pl.BlockSpec((XB, H), lambda i: (i, 0)),
            pl.BlockSpec((XB, H), lambda i: (i, 0)),
            pl.BlockSpec((1, H), lambda i: (0, 0)),
            pl.BlockSpec((1, H), lambda i: (0, 0)),
            pl.BlockSpec((1, H), lambda i: (0, 0)),
        ],
        out_specs=pl.BlockSpec((XB, H), lambda i: (i, 0)),
        out_shape=jax.ShapeDtypeStruct((N, H), _f32),
    )(o0, o1, b, g, bl)


def _pool_final_body(pp_ref, fw_ref, fb_ref, o_ref):
    m = jnp.max(pp_ref[...], axis=0)
    o_ref[...] = jnp.dot(m, fw_ref[...], preferred_element_type=_f32) \
        + fb_ref[...]


def _pool_final(pp, fw, fb):
    return pl.pallas_call(
        _pool_final_body,
        grid=(G // 8,),
        in_specs=[
            pl.BlockSpec((NW, 8, H), lambda i: (0, i, 0)),
            pl.BlockSpec((H, 1), lambda i: (0, 0)),
            pl.BlockSpec((1, 1), lambda i: (0, 0)),
        ],
        out_specs=pl.BlockSpec((8, 1), lambda i: (i, 0)),
        out_shape=jax.ShapeDtypeStruct((G, 1), _f32),
    )(pp, fw, fb)


def _ssum_reduce_body(pp_ref, o_ref):
    o_ref[...] = jnp.sum(pp_ref[...], axis=0, keepdims=True)[None]


def _ssum_reduce(pp):
    return pl.pallas_call(
        _ssum_reduce_body,
        grid=(8,),
        in_specs=[pl.BlockSpec((NW, NP // 8), lambda i: (0, i))],
        out_specs=pl.BlockSpec((1, 1, NP // 8), lambda i: (i, 0, 0)),
        out_shape=jax.ShapeDtypeStruct((8, 1, NP // 8), _f32),
    )(pp)


def _sc_mesh():
    return plsc.VectorSubcoreMesh(core_axis_name="c", subcore_axis_name="s")


_SC_PARAMS = pltpu.CompilerParams(needs_layout_passes=False)


@functools.partial(
    pl.kernel,
    out_type=[
        jax.ShapeDtypeStruct((EP,), _f32),
        jax.ShapeDtypeStruct((NW * NP,), _f32),
    ],
    mesh=_sc_mesh(),
    compiler_params=_SC_PARAMS,
    scratch_types=[
        pltpu.VMEM((NP,), _f32),
        pltpu.VMEM((NP,), _f32),
        pltpu.VMEM((NP,), _f32),
        pltpu.VMEM((16,), _f32),
        pltpu.VMEM((PW,), _i32),
        pltpu.VMEM((PW,), _i32),
        pltpu.VMEM((PW,), _f32),
        pltpu.VMEM((PW,), _f32),
    ],
)
def _sc_pass1(s_hbm, d_hbm, ce_hbm, src_hbm, dst_hbm, c_hbm,
              p_hbm, sspart_hbm,
              s_t, d_t, ssum_t, c_t, src_t, dst_t, ce_t, p_t):
    cid = lax.axis_index("c")
    sid = lax.axis_index("s")
    wid = cid * NS + sid
    base = wid * PW
    pltpu.sync_copy(s_hbm, s_t)
    pltpu.sync_copy(d_hbm, d_t)
    pltpu.sync_copy(c_hbm, c_t)
    pltpu.sync_copy(src_hbm.at[pl.ds(base, PW)], src_t)
    pltpu.sync_copy(dst_hbm.at[pl.ds(base, PW)], dst_t)
    pltpu.sync_copy(ce_hbm.at[pl.ds(base, PW)], ce_t)
    cv = c_t[...]

    def zero(i, carry):
        ssum_t[pl.ds(i * 16, 16)] = jnp.zeros((16,), _f32)
        return carry
    lax.fori_loop(0, NP // 16, zero, 0)

    def grp(j, carry):
        sl = pl.ds(j * 16, 16)
        si = src_t[sl]
        di = dst_t[sl]
        a = plsc.load_gather(s_t, [si]) + plsc.load_gather(d_t, [di]) \
            + ce_t[sl]
        a = jnp.where(a >= 0, a, a * 0.2)
        pv = jnp.exp(a - cv)
        p_t[sl] = pv
        plsc.addupdate_scatter(ssum_t, [di], pv)
        return carry
    lax.fori_loop(0, PW // 16, grp, 0)
    pltpu.sync_copy(p_t, p_hbm.at[pl.ds(base, PW)])
    pltpu.sync_copy(ssum_t, sspart_hbm.at[pl.ds(wid * NP, NP)])


@functools.partial(
    pl.kernel,
    out_type=[
        jax.ShapeDtypeStruct((EP,), _f32),
        jax.ShapeDtypeStruct((NC, NP, H), _f32),
    ],
    mesh=_sc_mesh(),
    compiler_params=_SC_PARAMS,
    scratch_types=[
        pltpu.VMEM((NP,), _f32),
        pltpu.VMEM((SUP,), _i32),
        pltpu.VMEM((SUP,), _i32),
        pltpu.VMEM((SUP,), _f32),
        pltpu.VMEM((SUP,), _f32),
        pltpu.VMEM((K2,), _i32),
        pltpu.VMEM((K2,), _i32),
        pltpu.VMEM((K2, H), _f32),
        pltpu.VMEM((K2, H), _f32),
        pltpu.VMEM_SHARED((NP, H), _f32),
        pltpu.SemaphoreType.DMA,
        pltpu.SemaphoreType.DMA,
        pltpu.SemaphoreType.DMA,
        pltpu.SemaphoreType.DMA,
    ],
)
def _sc_pass2(ssum_hbm, p_hbm, src_hbm, dst_hbm, h_hbm, z_hbm,
              w_hbm, outp_hbm,
              ssum_t, src_t, dst_t, p_t, w_t, dcA, dcB, rowsA, rowsB,
              acc_sh, sgA, sgB, ssA, ssB):
    cid = lax.axis_index("c")
    sid = lax.axis_index("s")
    wid = cid * NS + sid
    stripe = NP // NS
    CH = SUP // K2

    pltpu.sync_copy(ssum_hbm, ssum_t)
    pltpu.sync_copy(z_hbm, acc_sh.at[pl.ds(sid * stripe, stripe)])
    plsc.subcore_barrier()

    rbufs = (rowsA, rowsB)
    dbufs = (dcA, dcB)
    gsems = (sgA, sgB)
    ssems = (ssA, ssB)

    def do_super(s, carry):
        sb = wid * PW + s * SUP
        pltpu.sync_copy(src_hbm.at[pl.ds(sb, SUP)], src_t)
        pltpu.sync_copy(dst_hbm.at[pl.ds(sb, SUP)], dst_t)
        pltpu.sync_copy(p_hbm.at[pl.ds(sb, SUP)], p_t)
        scat = [None, None]
        gat = [pltpu.async_copy(h_hbm.at[src_t.at[pl.ds(0, K2)]],
                                rowsA, sgA), None]
        for c in range(CH):
            cur = c % 2
            nxt = 1 - cur
            if c >= 1:
                scat[nxt].wait()
            if c + 1 < CH:
                gat[nxt] = pltpu.async_copy(
                    h_hbm.at[src_t.at[pl.ds((c + 1) * K2, K2)]],
                    rbufs[nxt], gsems[nxt])
            gat[cur].wait()
            co = c * K2

            def wgrp(j, carry2, _co=co, _db=dbufs[cur]):
                sl16 = pl.ds(_co + j * 16, 16)
                di = dst_t[sl16]
                sv = plsc.load_gather(ssum_t, [di])
                w_t[sl16] = p_t[sl16] / (sv + 1e-16)
                _db[pl.ds(j * 16, 16)] = di
                return carry2
            lax.fori_loop(0, K2 // 16, wgrp, 0)

            def rowf(r2, carry2, _co=co, _rb=rbufs[cur]):
                r = r2 * 2
                wb0 = plsc.load_gather(w_t, [jnp.full((16,), _co, _i32) + r])
                wb1 = plsc.load_gather(
                    w_t, [jnp.full((16,), _co + 1, _i32) + r])
                for cg in range(H // 16):
                    cs = pl.ds(cg * 16, 16)
                    _rb[r, cs] = _rb[r, cs] * wb0
                    _rb[r + 1, cs] = _rb[r + 1, cs] * wb1
                return carry2
            lax.fori_loop(0, K2 // 2, rowf, 0)
            scat[cur] = pltpu.async_copy(rbufs[cur], acc_sh.at[dbufs[cur]],
                                         ssems[cur], add=True)
        scat[(CH - 1) % 2].wait()
        pltpu.sync_copy(w_t, w_hbm.at[pl.ds(sb, SUP)])
        return carry
    lax.fori_loop(0, PW // SUP, do_super, 0)

    plsc.subcore_barrier()
    for i in range(stripe // 64):
        row0 = sid * stripe + i * 64
        pltpu.sync_copy(acc_sh.at[pl.ds(row0, 64)], rowsA.at[pl.ds(0, 64)])
        pltpu.sync_copy(rowsA.at[pl.ds(0, 64)],
                        outp_hbm.at[cid, pl.ds(row0, 64)])


@functools.partial(
    pl.kernel,
    out_type=jax.ShapeDtypeStruct((NW, G, H), _f32),
    mesh=_sc_mesh(),
    compiler_params=_SC_PARAMS,
    scratch_types=[
        pltpu.VMEM((G, H), _f32),
        pltpu.VMEM((64, H), _f32),
        pltpu.VMEM((NP // NW,), _i32),
    ],
)
def _sc_pool(hr_hbm, bat_hbm, z_hbm, pool_hbm, tbl_t, rows_t, bat_t):
    cid = lax.axis_index("c")
    sid = lax.axis_index("s")
    wid = cid * NS + sid
    rpw = NP // NW
    pltpu.sync_copy(z_hbm.at[pl.ds(0, G)], tbl_t)
    pltpu.sync_copy(bat_hbm.at[pl.ds(wid * rpw, rpw)], bat_t)
    colio = lax.iota(_i32, 16)

    def chunk(i, carry):
        pltpu.sync_copy(hr_hbm.at[pl.ds(wid * rpw + i * 64, 64)], rows_t)

        def row(r, carry2):
            gv = plsc.load_gather(bat_t, [jnp.full((16,), i * 64, _i32) + r])
            ri = jnp.full((16,), r, _i32)
            for c in range(H // 16):
                ci = colio + (c * 16)
                v = plsc.load_gather(rows_t, [ri, ci])
                cur = plsc.load_gather(tbl_t, [gv, ci])
                plsc.store_scatter(tbl_t, [gv, ci], jnp.maximum(cur, v))
            return carry2
        lax.fori_loop(0, 64, row, 0)
        return carry
    lax.fori_loop(0, rpw // 64, chunk, 0)
    pltpu.sync_copy(tbl_t, pool_hbm.at[wid])


def kernel(x, edge_index, edge_attr, batch, W1, as1, ad1, ae1, We1, b1,
           W2, as2, ad2, ae2, We2, b2, ln_g, ln_b, fc_w, fc_b):
    loop = jnp.arange(N, dtype=_i32)
    padi = jnp.zeros((EP - ET,), _i32)
    src = jnp.concatenate([edge_index[0].astype(_i32), loop, padi])
    dst = jnp.concatenate([edge_index[1].astype(_i32), loop, padi])

    ce1e, ce2e, est = _edge_pre(edge_attr.reshape(E // 8, 128),
                                We1.T, ae1.reshape(1, H),
                                We2.T, ae2.reshape(1, H))
    mean1 = jnp.sum(est[:, 0, 0]) / E
    mean2 = jnp.sum(est[:, 0, 2]) / E
    maxce1 = jnp.maximum(jnp.max(est[:, 0, 1]), mean1)
    maxce2 = jnp.maximum(jnp.max(est[:, 0, 3]), mean2)
    padf = jnp.full((EP - ET,), -1e30, _f32)
    ce1 = jnp.concatenate([ce1e.reshape(E), jnp.full((N,), mean1, _f32), padf])
    ce2 = jnp.concatenate([ce2e.reshape(E), jnp.full((N,), mean2, _f32), padf])

    zrows = jnp.zeros((NP // NS, H), _f32)
    b1r = b1.reshape(1, H)
    b2r = b2.reshape(1, H)
    gr = ln_g.reshape(1, H)
    blr = ln_b.reshape(1, H)

    def _padn(v):
        return jnp.concatenate([v.reshape(-1), jnp.zeros((NP - N,), _f32)])

    h1, s1, d1, nst1 = _node_pre(x, W1, as1.reshape(H, 1), ad1.reshape(H, 1))
    c1 = jnp.maximum(jnp.max(nst1[:, 0, 0]) + jnp.max(nst1[:, 0, 1]) + maxce1,
                     0.0)
    p1, sspart1 = _sc_pass1(_padn(s1), _padn(d1), ce1, src, dst,
                            jnp.full((16,), c1, _f32))
    ssum1 = _ssum_reduce(sspart1.reshape(NW, NP)).reshape(NP)
    w1, outp1 = _sc_pass2(ssum1, p1, src, dst, h1, zrows)

    h2, s2, d2, nst2 = _post1(outp1[0, :N], outp1[1, :N], b1r, gr, blr,
                              W2, as2.reshape(H, 1), ad2.reshape(H, 1))
    c2 = jnp.maximum(jnp.max(nst2[:, 0, 0]) + jnp.max(nst2[:, 0, 1]) + maxce2,
                     0.0)
    p2, sspart2 = _sc_pass1(_padn(s2), _padn(d2), ce2, src, dst,
                            jnp.full((16,), c2, _f32))
    ssum2 = _ssum_reduce(sspart2.reshape(NW, NP)).reshape(NP)
    w2, outp2 = _sc_pass2(ssum2, p2, src, dst, h2, zrows)

    hrel = _post2(outp2[0, :N], outp2[1, :N], b2r, gr, blr)
    hrelp = jnp.concatenate([hrel, jnp.zeros((NP - N, H), _f32)], axis=0)
    batp = jnp.concatenate([batch.astype(_i32), jnp.zeros((NP - N,), _i32)])
    pool = _sc_pool(hrelp, batp, zrows)
    out = _pool_final(pool, fc_w, fc_b.reshape(1, 1))
    return (out.reshape(-1), w1[:ET], w2[:ET])
